# Initial kernel scaffold; baseline (speedup 1.0000x reference)
import jax
import jax.numpy as jnp
from jax.experimental import pallas as pl


def kernel(x2, edge_index2, batch, lin0_W, lin0_b, gcn_W, gcn_b, gru_Wih, gru_Whh, gru_bih, gru_bhh, lstm_Wih, lstm_Whh, lstm_bih, lstm_bhh):
    raise NotImplementedError("write your pallas kernel here")


# speedup vs baseline: 11.6718x; 11.6718x over previous
import functools

import jax
import jax.numpy as jnp
from jax import lax
from jax.experimental import pallas as pl
from jax.experimental.pallas import tpu as pltpu
from jax.experimental.pallas import tpu_sc as plsc

N = 10000
E = 320000
D = 128
B = 64

NC = 2
NS = 16
NW = NC * NS
NP = 10240
RPW = NP // NS
EW = E // NW
K = 80
NCHUNK = EW // K

R = 1024
NB = NP // R

_f32 = jnp.float32


def _sc_deg_body(dst_hbm, degp_hbm, idx_v, ones_v, zero_v, acc_s, sem):
    c = lax.axis_index("c")
    s = lax.axis_index("s")
    w = c * NS + s

    one16 = jnp.ones((16,), _f32)
    zero16 = jnp.zeros((16,), _f32)
    for i in range(K // 16):
        ones_v[pl.ds(i * 16, 16)] = one16

    def zbody(i, carry):
        zero_v[pl.ds(i * 16, 16)] = zero16
        return carry

    lax.fori_loop(0, RPW // 16, zbody, 0)

    pltpu.sync_copy(zero_v, acc_s.at[pl.ds(s * RPW, RPW)])
    plsc.subcore_barrier()

    def body(j, carry):
        base = pl.multiple_of(w * EW + j * K, 8)
        pltpu.sync_copy(dst_hbm.at[pl.ds(base, K)], idx_v)
        pltpu.sync_copy(ones_v, acc_s.at[idx_v], add=True)
        return carry

    lax.fori_loop(0, NCHUNK, body, 0)
    plsc.subcore_barrier()

    pltpu.sync_copy(acc_s.at[pl.ds(s * RPW, RPW)],
                    degp_hbm.at[pl.ds(c * NP + s * RPW, RPW)])


def _sc_degrees(dst):
    mesh = plsc.VectorSubcoreMesh(core_axis_name="c", subcore_axis_name="s")
    fn = pl.kernel(
        _sc_deg_body,
        out_type=jax.ShapeDtypeStruct((NC * NP,), _f32),
        mesh=mesh,
        scratch_types=[
            pltpu.VMEM((K,), jnp.int32),
            pltpu.VMEM((K,), _f32),
            pltpu.VMEM((RPW,), _f32),
            pltpu.VMEM_SHARED((NP,), _f32),
            pltpu.SemaphoreType.DMA,
        ],
    )
    return fn(dst)


def _sc_agg_body(y_hbm, src_hbm, dst_hbm, aggp_hbm,
                 src_v, dst_v, rows_v, acc_s, sem):
    c = lax.axis_index("c")
    s = lax.axis_index("s")
    w = c * NS + s

    zero16 = jnp.zeros((16,), _f32)

    def zbody(i, carry):
        for cc in range(D // 16):
            rows_v[i, pl.ds(cc * 16, 16)] = zero16
        return carry

    lax.fori_loop(0, K, zbody, 0)
    for jj in range(RPW // K):
        pltpu.sync_copy(rows_v, acc_s.at[pl.ds(s * RPW + jj * K, K)])
    plsc.subcore_barrier()

    def body(j, carry):
        base = pl.multiple_of(w * EW + j * K, 8)
        pltpu.sync_copy(src_hbm.at[pl.ds(base, K)], src_v)
        pltpu.async_copy(y_hbm.at[src_v], rows_v, sem).wait()
        pltpu.sync_copy(dst_hbm.at[pl.ds(base, K)], dst_v)
        pltpu.sync_copy(rows_v, acc_s.at[dst_v], add=True)
        return carry

    lax.fori_loop(0, NCHUNK, body, 0)
    plsc.subcore_barrier()

    pltpu.sync_copy(acc_s.at[pl.ds(s * RPW, RPW)],
                    aggp_hbm.at[pl.ds(c * NP + s * RPW, RPW)])


def _sc_aggregate(y, src, dst):
    mesh = plsc.VectorSubcoreMesh(core_axis_name="c", subcore_axis_name="s")
    fn = pl.kernel(
        _sc_agg_body,
        out_type=jax.ShapeDtypeStruct((NC * NP, D), _f32),
        mesh=mesh,
        scratch_types=[
            pltpu.VMEM((K,), jnp.int32),
            pltpu.VMEM((K,), jnp.int32),
            pltpu.VMEM((K, D), _f32),
            pltpu.VMEM_SHARED((NP, D), _f32),
            pltpu.SemaphoreType.DMA,
        ],
    )
    return fn(y, src, dst)


def _mm_t(a, w):
    return lax.dot_general(a, w, (((1,), (1,)), ((), ())),
                           preferred_element_type=_f32)


def _tc_lin0_body(x_ref, degp_ref, lw_ref, lb_ref, gw_ref, h_ref, y_ref):
    h = jnp.maximum(_mm_t(x_ref[...], lw_ref[...]) + lb_ref[...], 0.0)
    deg = degp_ref[0, :] + degp_ref[1, :] + 1.0
    dinv = lax.rsqrt(deg)
    h_ref[...] = h
    y_ref[...] = dinv[:, None] * _mm_t(h, gw_ref[...])


def _tc_lin0(xp, degp, lin0_W, lin0_b, gcn_W):
    return pl.pallas_call(
        _tc_lin0_body,
        grid=(NB,),
        in_specs=[
            pl.BlockSpec((R, D), lambda i: (i, 0)),
            pl.BlockSpec((NC, R), lambda i: (0, i)),
            pl.BlockSpec((D, D), lambda i: (0, 0)),
            pl.BlockSpec((D,), lambda i: (0,)),
            pl.BlockSpec((D, D), lambda i: (0, 0)),
        ],
        out_specs=[
            pl.BlockSpec((R, D), lambda i: (i, 0)),
            pl.BlockSpec((R, D), lambda i: (i, 0)),
        ],
        out_shape=[
            jax.ShapeDtypeStruct((NP, D), _f32),
            jax.ShapeDtypeStruct((NP, D), _f32),
        ],
    )(xp, degp, lin0_W, lin0_b, gcn_W)


def _tc_round_body(aggp_ref, y_ref, h_ref, degp_ref, gb_ref,
                   wih_ref, whh_ref, bih_ref, bhh_ref, gw_ref,
                   hn_ref, yn_ref):
    agg = aggp_ref[0, :, :] + aggp_ref[1, :, :]
    deg = degp_ref[0, :] + degp_ref[1, :] + 1.0
    dinv = lax.rsqrt(deg)
    m = jnp.maximum(dinv[:, None] * (agg + y_ref[...]) + gb_ref[...], 0.0)
    h = h_ref[...]
    gi = _mm_t(m, wih_ref[...]) + bih_ref[...]
    gh = _mm_t(h, whh_ref[...]) + bhh_ref[...]
    r = jax.nn.sigmoid(gi[:, :D] + gh[:, :D])
    z = jax.nn.sigmoid(gi[:, D:2 * D] + gh[:, D:2 * D])
    n = jnp.tanh(gi[:, 2 * D:] + r * gh[:, 2 * D:])
    hn = (1.0 - z) * n + z * h
    hn_ref[...] = hn
    yn_ref[...] = dinv[:, None] * _mm_t(hn, gw_ref[...])


def _tc_round(aggp, y, h, degp, gcn_b, gru_Wih, gru_Whh, gru_bih, gru_bhh,
              gcn_W):
    return pl.pallas_call(
        _tc_round_body,
        grid=(NB,),
        in_specs=[
            pl.BlockSpec((NC, R, D), lambda i: (0, i, 0)),
            pl.BlockSpec((R, D), lambda i: (i, 0)),
            pl.BlockSpec((R, D), lambda i: (i, 0)),
            pl.BlockSpec((NC, R), lambda i: (0, i)),
            pl.BlockSpec((D,), lambda i: (0,)),
            pl.BlockSpec((3 * D, D), lambda i: (0, 0)),
            pl.BlockSpec((3 * D, D), lambda i: (0, 0)),
            pl.BlockSpec((3 * D,), lambda i: (0,)),
            pl.BlockSpec((3 * D,), lambda i: (0,)),
            pl.BlockSpec((D, D), lambda i: (0, 0)),
        ],
        out_specs=[
            pl.BlockSpec((R, D), lambda i: (i, 0)),
            pl.BlockSpec((R, D), lambda i: (i, 0)),
        ],
        out_shape=[
            jax.ShapeDtypeStruct((NP, D), _f32),
            jax.ShapeDtypeStruct((NP, D), _f32),
        ],
    )(aggp, y, h, degp, gcn_b, gru_Wih, gru_Whh, gru_bih, gru_bhh, gcn_W)


def _tc_set2set_body(x_ref, batch_ref, wih_ref, whh_ref, bih_ref, bhh_ref,
                     q_ref):
    x = x_ref[...]
    bvec = batch_ref[...]
    seg = lax.broadcasted_iota(jnp.int32, (N, B), 1)
    S = (bvec == seg).astype(_f32)

    h_l = jnp.zeros((B, D), _f32)
    c_l = jnp.zeros((B, D), _f32)
    q_star = jnp.zeros((B, 2 * D), _f32)
    for _ in range(3):
        gates = (_mm_t(q_star, wih_ref[...]) + bih_ref[...]
                 + _mm_t(h_l, whh_ref[...]) + bhh_ref[...])
        gi = gates[:, :D]
        gf = gates[:, D:2 * D]
        gg = gates[:, 2 * D:3 * D]
        go = gates[:, 3 * D:]
        c_l = jax.nn.sigmoid(gf) * c_l + jax.nn.sigmoid(gi) * jnp.tanh(gg)
        h_l = jax.nn.sigmoid(go) * jnp.tanh(c_l)
        qg = lax.dot_general(S, h_l, (((1,), (0,)), ((), ())),
                             preferred_element_type=_f32)
        e = jnp.sum(x * qg, axis=1, keepdims=True)
        em = jnp.where(S > 0.5, e, -1e30)
        mseg = jnp.max(em, axis=0, keepdims=True)
        mg = lax.dot_general(S, mseg, (((1,), (1,)), ((), ())),
                             preferred_element_type=_f32)
        e2 = jnp.exp(e - mg)
        sseg = lax.dot_general(S, e2, (((0,), (0,)), ((), ())),
                               preferred_element_type=_f32)
        sg = lax.dot_general(S, sseg, (((1,), (0,)), ((), ())),
                             preferred_element_type=_f32)
        a = e2 / (sg + 1e-16)
        r = lax.dot_general(S, a * x, (((0,), (0,)), ((), ())),
                            preferred_element_type=_f32)
        q_star = jnp.concatenate([h_l, r], axis=1)
    q_ref[...] = q_star


def _tc_set2set(x, batch2d, lstm_Wih, lstm_Whh, lstm_bih, lstm_bhh):
    return pl.pallas_call(
        _tc_set2set_body,
        out_shape=jax.ShapeDtypeStruct((B, 2 * D), _f32),
    )(x, batch2d, lstm_Wih, lstm_Whh, lstm_bih, lstm_bhh)


def kernel(x2, edge_index2, batch, lin0_W, lin0_b, gcn_W, gcn_b,
           gru_Wih, gru_Whh, gru_bih, gru_bhh,
           lstm_Wih, lstm_Whh, lstm_bih, lstm_bhh):
    src = edge_index2[0]
    dst = edge_index2[1]

    degp = _sc_degrees(dst).reshape(NC, NP)

    xp = jnp.pad(x2.astype(_f32), ((0, NP - N), (0, 0)))
    h, y = _tc_lin0(xp, degp, lin0_W, lin0_b, gcn_W)

    for _ in range(3):
        aggp = _sc_aggregate(y, src, dst).reshape(NC, NP, D)
        h, y = _tc_round(aggp, y, h, degp, gcn_b,
                         gru_Wih, gru_Whh, gru_bih, gru_bhh, gcn_W)

    feat_last = h[:N]
    batch2d = batch.reshape(N, 1)
    q_star = _tc_set2set(feat_last, batch2d,
                         lstm_Wih, lstm_Whh, lstm_bih, lstm_bhh)
    return (q_star, feat_last)


# speedup vs baseline: 17.8370x; 1.5282x over previous
import jax
import jax.numpy as jnp
from jax import lax
from jax.experimental import pallas as pl
from jax.experimental.pallas import tpu as pltpu
from jax.experimental.pallas import tpu_sc as plsc

N = 10000
E = 320000
D = 128
B = 64

NC = 2
NS = 16
NW = NC * NS
NP = 10240
RPW = NP // NS
EW = E // NW
K = 80
NCHUNK = EW // K

R = 1024
NB = NP // R

_f32 = jnp.float32


def _sc_deg_body(dst_hbm, degp_hbm, idx_v, ones_v, zero_v, acc_s, sem):
    c = lax.axis_index("c")
    s = lax.axis_index("s")
    w = c * NS + s

    one16 = jnp.ones((16,), _f32)
    zero16 = jnp.zeros((16,), _f32)
    for i in range(K // 16):
        ones_v[pl.ds(i * 16, 16)] = one16

    def zbody(i, carry):
        zero_v[pl.ds(i * 16, 16)] = zero16
        return carry

    lax.fori_loop(0, RPW // 16, zbody, 0)

    pltpu.sync_copy(zero_v, acc_s.at[pl.ds(s * RPW, RPW)])
    plsc.subcore_barrier()

    def body(j, carry):
        base = pl.multiple_of(w * EW + j * K, 8)
        pltpu.sync_copy(dst_hbm.at[pl.ds(base, K)], idx_v)
        pltpu.sync_copy(ones_v, acc_s.at[idx_v], add=True)
        return carry

    lax.fori_loop(0, NCHUNK, body, 0)
    plsc.subcore_barrier()

    pltpu.sync_copy(acc_s.at[pl.ds(s * RPW, RPW)],
                    degp_hbm.at[pl.ds(c * NP + s * RPW, RPW)])


def _sc_degrees(dst):
    mesh = plsc.VectorSubcoreMesh(core_axis_name="c", subcore_axis_name="s")
    fn = pl.kernel(
        _sc_deg_body,
        out_type=jax.ShapeDtypeStruct((NC * NP,), _f32),
        mesh=mesh,
        scratch_types=[
            pltpu.VMEM((K,), jnp.int32),
            pltpu.VMEM((K,), _f32),
            pltpu.VMEM((RPW,), _f32),
            pltpu.VMEM_SHARED((NP,), _f32),
            pltpu.SemaphoreType.DMA,
        ],
    )
    return fn(dst)


def _sc_agg_body(y_hbm, src_hbm, dst_hbm, aggp_hbm,
                 sa, sb, da, db, rows_a, rows_b, acc_s, gsa, gsb):
    c = lax.axis_index("c")
    s = lax.axis_index("s")
    w = c * NS + s

    zero16 = jnp.zeros((16,), _f32)

    def zbody(i, carry):
        for cc in range(D // 16):
            rows_a[i, pl.ds(cc * 16, 16)] = zero16
        return carry

    lax.fori_loop(0, K, zbody, 0)
    for jj in range(RPW // K):
        pltpu.sync_copy(rows_a, acc_s.at[pl.ds(s * RPW + jj * K, K)])
    plsc.subcore_barrier()

    def issue(j, sidx, rbuf, sem):
        base = pl.multiple_of(w * EW + j * K, 8)
        pltpu.sync_copy(src_hbm.at[pl.ds(base, K)], sidx)
        pltpu.async_copy(y_hbm.at[sidx], rbuf, sem)

    def finish(j, sidx, didx, rbuf, sem):
        base = pl.multiple_of(w * EW + j * K, 8)
        pltpu.make_async_copy(y_hbm.at[sidx], rbuf, sem).wait()
        pltpu.sync_copy(dst_hbm.at[pl.ds(base, K)], didx)
        pltpu.sync_copy(rbuf, acc_s.at[didx], add=True)

    issue(0, sa, rows_a, gsa)

    def body(g, carry):
        j = 2 * g
        issue(j + 1, sb, rows_b, gsb)
        finish(j, sa, da, rows_a, gsa)
        issue(j + 2, sa, rows_a, gsa)
        finish(j + 1, sb, db, rows_b, gsb)
        return carry

    lax.fori_loop(0, (NCHUNK - 1) // 2, body, 0)
    finish(NCHUNK - 1, sa, da, rows_a, gsa)
    plsc.subcore_barrier()

    pltpu.sync_copy(acc_s.at[pl.ds(s * RPW, RPW)],
                    aggp_hbm.at[pl.ds(c * NP + s * RPW, RPW)])


def _sc_aggregate(y, src, dst):
    mesh = plsc.VectorSubcoreMesh(core_axis_name="c", subcore_axis_name="s")
    fn = pl.kernel(
        _sc_agg_body,
        out_type=jax.ShapeDtypeStruct((NC * NP, D), _f32),
        mesh=mesh,
        scratch_types=[
            pltpu.VMEM((K,), jnp.int32),
            pltpu.VMEM((K,), jnp.int32),
            pltpu.VMEM((K,), jnp.int32),
            pltpu.VMEM((K,), jnp.int32),
            pltpu.VMEM((K, D), _f32),
            pltpu.VMEM((K, D), _f32),
            pltpu.VMEM_SHARED((NP, D), _f32),
            pltpu.SemaphoreType.DMA,
            pltpu.SemaphoreType.DMA,
        ],
    )
    return fn(y, src, dst)


def _mm_t(a, w):
    return lax.dot_general(a, w, (((1,), (1,)), ((), ())),
                           preferred_element_type=_f32)


def _tc_lin0_body(x_ref, degp_ref, lw_ref, lb_ref, gw_ref, h_ref, y_ref):
    h = jnp.maximum(_mm_t(x_ref[...], lw_ref[...]) + lb_ref[...], 0.0)
    deg = degp_ref[0, :] + degp_ref[1, :] + 1.0
    dinv = lax.rsqrt(deg)
    h_ref[...] = h
    y_ref[...] = dinv[:, None] * _mm_t(h, gw_ref[...])


def _tc_lin0(xp, degp, lin0_W, lin0_b, gcn_W):
    return pl.pallas_call(
        _tc_lin0_body,
        grid=(NB,),
        in_specs=[
            pl.BlockSpec((R, D), lambda i: (i, 0)),
            pl.BlockSpec((NC, R), lambda i: (0, i)),
            pl.BlockSpec((D, D), lambda i: (0, 0)),
            pl.BlockSpec((D,), lambda i: (0,)),
            pl.BlockSpec((D, D), lambda i: (0, 0)),
        ],
        out_specs=[
            pl.BlockSpec((R, D), lambda i: (i, 0)),
            pl.BlockSpec((R, D), lambda i: (i, 0)),
        ],
        out_shape=[
            jax.ShapeDtypeStruct((NP, D), _f32),
            jax.ShapeDtypeStruct((NP, D), _f32),
        ],
    )(xp, degp, lin0_W, lin0_b, gcn_W)


def _tc_round_body(aggp_ref, y_ref, h_ref, degp_ref, gb_ref,
                   wih_ref, whh_ref, bih_ref, bhh_ref, gw_ref,
                   hn_ref, yn_ref):
    agg = aggp_ref[0, :, :] + aggp_ref[1, :, :]
    deg = degp_ref[0, :] + degp_ref[1, :] + 1.0
    dinv = lax.rsqrt(deg)
    m = jnp.maximum(dinv[:, None] * (agg + y_ref[...]) + gb_ref[...], 0.0)
    h = h_ref[...]
    gi = _mm_t(m, wih_ref[...]) + bih_ref[...]
    gh = _mm_t(h, whh_ref[...]) + bhh_ref[...]
    r = jax.nn.sigmoid(gi[:, :D] + gh[:, :D])
    z = jax.nn.sigmoid(gi[:, D:2 * D] + gh[:, D:2 * D])
    n = jnp.tanh(gi[:, 2 * D:] + r * gh[:, 2 * D:])
    hn = (1.0 - z) * n + z * h
    hn_ref[...] = hn
    yn_ref[...] = dinv[:, None] * _mm_t(hn, gw_ref[...])


def _tc_round(aggp, y, h, degp, gcn_b, gru_Wih, gru_Whh, gru_bih, gru_bhh,
              gcn_W):
    return pl.pallas_call(
        _tc_round_body,
        grid=(NB,),
        in_specs=[
            pl.BlockSpec((NC, R, D), lambda i: (0, i, 0)),
            pl.BlockSpec((R, D), lambda i: (i, 0)),
            pl.BlockSpec((R, D), lambda i: (i, 0)),
            pl.BlockSpec((NC, R), lambda i: (0, i)),
            pl.BlockSpec((D,), lambda i: (0,)),
            pl.BlockSpec((3 * D, D), lambda i: (0, 0)),
            pl.BlockSpec((3 * D, D), lambda i: (0, 0)),
            pl.BlockSpec((3 * D,), lambda i: (0,)),
            pl.BlockSpec((3 * D,), lambda i: (0,)),
            pl.BlockSpec((D, D), lambda i: (0, 0)),
        ],
        out_specs=[
            pl.BlockSpec((R, D), lambda i: (i, 0)),
            pl.BlockSpec((R, D), lambda i: (i, 0)),
        ],
        out_shape=[
            jax.ShapeDtypeStruct((NP, D), _f32),
            jax.ShapeDtypeStruct((NP, D), _f32),
        ],
    )(aggp, y, h, degp, gcn_b, gru_Wih, gru_Whh, gru_bih, gru_bhh, gcn_W)


def _tc_set2set_body(x_ref, batch_ref, wih_ref, whh_ref, bih_ref, bhh_ref,
                     q_ref):
    x = x_ref[...]
    bvec = batch_ref[...]
    seg = lax.broadcasted_iota(jnp.int32, (N, B), 1)
    S = (bvec == seg).astype(_f32)

    h_l = jnp.zeros((B, D), _f32)
    c_l = jnp.zeros((B, D), _f32)
    q_star = jnp.zeros((B, 2 * D), _f32)
    for _ in range(3):
        gates = (_mm_t(q_star, wih_ref[...]) + bih_ref[...]
                 + _mm_t(h_l, whh_ref[...]) + bhh_ref[...])
        gi = gates[:, :D]
        gf = gates[:, D:2 * D]
        gg = gates[:, 2 * D:3 * D]
        go = gates[:, 3 * D:]
        c_l = jax.nn.sigmoid(gf) * c_l + jax.nn.sigmoid(gi) * jnp.tanh(gg)
        h_l = jax.nn.sigmoid(go) * jnp.tanh(c_l)
        qg = lax.dot_general(S, h_l, (((1,), (0,)), ((), ())),
                             preferred_element_type=_f32)
        e = jnp.sum(x * qg, axis=1, keepdims=True)
        em = jnp.where(S > 0.5, e, -1e30)
        mseg = jnp.max(em, axis=0, keepdims=True)
        mg = lax.dot_general(S, mseg, (((1,), (1,)), ((), ())),
                             preferred_element_type=_f32)
        e2 = jnp.exp(e - mg)
        sseg = lax.dot_general(S, e2, (((0,), (0,)), ((), ())),
                               preferred_element_type=_f32)
        sg = lax.dot_general(S, sseg, (((1,), (0,)), ((), ())),
                             preferred_element_type=_f32)
        a = e2 / (sg + 1e-16)
        r = lax.dot_general(S, a * x, (((0,), (0,)), ((), ())),
                            preferred_element_type=_f32)
        q_star = jnp.concatenate([h_l, r], axis=1)
    q_ref[...] = q_star


def _tc_set2set(x, batch2d, lstm_Wih, lstm_Whh, lstm_bih, lstm_bhh):
    return pl.pallas_call(
        _tc_set2set_body,
        out_shape=jax.ShapeDtypeStruct((B, 2 * D), _f32),
    )(x, batch2d, lstm_Wih, lstm_Whh, lstm_bih, lstm_bhh)


def kernel(x2, edge_index2, batch, lin0_W, lin0_b, gcn_W, gcn_b,
           gru_Wih, gru_Whh, gru_bih, gru_bhh,
           lstm_Wih, lstm_Whh, lstm_bih, lstm_bhh):
    src = edge_index2[0]
    dst = edge_index2[1]

    degp = _sc_degrees(dst).reshape(NC, NP)

    xp = jnp.pad(x2.astype(_f32), ((0, NP - N), (0, 0)))
    h, y = _tc_lin0(xp, degp, lin0_W, lin0_b, gcn_W)

    for _ in range(3):
        aggp = _sc_aggregate(y, src, dst).reshape(NC, NP, D)
        h, y = _tc_round(aggp, y, h, degp, gcn_b,
                         gru_Wih, gru_Whh, gru_bih, gru_bhh, gcn_W)

    feat_last = h[:N]
    batch2d = batch.reshape(N, 1)
    q_star = _tc_set2set(feat_last, batch2d,
                         lstm_Wih, lstm_Whh, lstm_bih, lstm_bhh)
    return (q_star, feat_last)


# speedup vs baseline: 21.2040x; 1.1888x over previous
import jax
import jax.numpy as jnp
from jax import lax
from jax.experimental import pallas as pl
from jax.experimental.pallas import tpu as pltpu
from jax.experimental.pallas import tpu_sc as plsc

N = 10000
E = 320000
D = 128
B = 64

NC = 2
NS = 16
NW = NC * NS
NP = 10240
RPW = NP // NS
EW = E // NW
K = 80
NCHUNK = EW // K

R = 1024
NB = NP // R

_f32 = jnp.float32


def _sc_deg_body(dst_hbm, degp_hbm, idx_v, ones_v, zero_v, acc_s, sem):
    c = lax.axis_index("c")
    s = lax.axis_index("s")
    w = c * NS + s

    one16 = jnp.ones((16,), _f32)
    zero16 = jnp.zeros((16,), _f32)
    for i in range(K // 16):
        ones_v[pl.ds(i * 16, 16)] = one16

    def zbody(i, carry):
        zero_v[pl.ds(i * 16, 16)] = zero16
        return carry

    lax.fori_loop(0, RPW // 16, zbody, 0)

    pltpu.sync_copy(zero_v, acc_s.at[pl.ds(s * RPW, RPW)])
    plsc.subcore_barrier()

    def body(j, carry):
        base = pl.multiple_of(w * EW + j * K, 8)
        pltpu.sync_copy(dst_hbm.at[pl.ds(base, K)], idx_v)
        pltpu.sync_copy(ones_v, acc_s.at[idx_v], add=True)
        return carry

    lax.fori_loop(0, NCHUNK, body, 0)
    plsc.subcore_barrier()

    pltpu.sync_copy(acc_s.at[pl.ds(s * RPW, RPW)],
                    degp_hbm.at[pl.ds(c * NP + s * RPW, RPW)])


def _sc_degrees(dst):
    mesh = plsc.VectorSubcoreMesh(core_axis_name="c", subcore_axis_name="s")
    fn = pl.kernel(
        _sc_deg_body,
        out_type=jax.ShapeDtypeStruct((NC * NP,), _f32),
        mesh=mesh,
        scratch_types=[
            pltpu.VMEM((K,), jnp.int32),
            pltpu.VMEM((K,), _f32),
            pltpu.VMEM((RPW,), _f32),
            pltpu.VMEM_SHARED((NP,), _f32),
            pltpu.SemaphoreType.DMA,
        ],
    )
    return fn(dst)


def _sc_agg_body(y_hbm, src_hbm, dst_hbm, aggp_hbm,
                 sa, sb, da, db, rows_a, rows_b, acc_s, gsa, gsb, ea, eb):
    c = lax.axis_index("c")
    s = lax.axis_index("s")
    w = c * NS + s

    zero16 = jnp.zeros((16,), _f32)

    def zbody(i, carry):
        for cc in range(D // 16):
            rows_a[i, pl.ds(cc * 16, 16)] = zero16
        return carry

    lax.fori_loop(0, K, zbody, 0)
    for jj in range(RPW // K):
        pltpu.sync_copy(rows_a, acc_s.at[pl.ds(s * RPW + jj * K, K)])
    plsc.subcore_barrier()

    def ebase(j):
        return pl.multiple_of(w * EW + j * K, 8)

    def issue(j, sidx, rbuf, gsem, didx, dsem):
        pltpu.sync_copy(src_hbm.at[pl.ds(ebase(j), K)], sidx)
        pltpu.async_copy(y_hbm.at[sidx], rbuf, gsem)
        pltpu.async_copy(dst_hbm.at[pl.ds(ebase(j), K)], didx, dsem)

    def finish(j, sidx, rbuf, gsem, didx, dsem):
        pltpu.make_async_copy(y_hbm.at[sidx], rbuf, gsem).wait()
        pltpu.make_async_copy(dst_hbm.at[pl.ds(ebase(j), K)], didx,
                              dsem).wait()
        pltpu.sync_copy(rbuf, acc_s.at[didx], add=True)

    issue(0, sa, rows_a, gsa, da, ea)

    def body(g, carry):
        j = 2 * g
        issue(j + 1, sb, rows_b, gsb, db, eb)
        finish(j, sa, rows_a, gsa, da, ea)
        issue(j + 2, sa, rows_a, gsa, da, ea)
        finish(j + 1, sb, rows_b, gsb, db, eb)
        return carry

    lax.fori_loop(0, (NCHUNK - 1) // 2, body, 0)
    finish(NCHUNK - 1, sa, rows_a, gsa, da, ea)
    plsc.subcore_barrier()

    pltpu.sync_copy(acc_s.at[pl.ds(s * RPW, RPW)],
                    aggp_hbm.at[pl.ds(c * NP + s * RPW, RPW)])


def _sc_aggregate(y, src, dst):
    mesh = plsc.VectorSubcoreMesh(core_axis_name="c", subcore_axis_name="s")
    fn = pl.kernel(
        _sc_agg_body,
        out_type=jax.ShapeDtypeStruct((NC * NP, D), _f32),
        mesh=mesh,
        scratch_types=[
            pltpu.VMEM((K,), jnp.int32),
            pltpu.VMEM((K,), jnp.int32),
            pltpu.VMEM((K,), jnp.int32),
            pltpu.VMEM((K,), jnp.int32),
            pltpu.VMEM((K, D), _f32),
            pltpu.VMEM((K, D), _f32),
            pltpu.VMEM_SHARED((NP, D), _f32),
            pltpu.SemaphoreType.DMA,
            pltpu.SemaphoreType.DMA,
            pltpu.SemaphoreType.DMA,
            pltpu.SemaphoreType.DMA,
        ],
    )
    return fn(y, src, dst)


def _mm_t(a, w):
    return lax.dot_general(a, w, (((1,), (1,)), ((), ())),
                           preferred_element_type=_f32)


def _tc_lin0_body(x_ref, degp_ref, lw_ref, lb_ref, gw_ref, h_ref, y_ref):
    h = jnp.maximum(_mm_t(x_ref[...], lw_ref[...]) + lb_ref[...], 0.0)
    deg = degp_ref[0, :] + degp_ref[1, :] + 1.0
    dinv = lax.rsqrt(deg)
    h_ref[...] = h
    y_ref[...] = dinv[:, None] * _mm_t(h, gw_ref[...])


def _tc_lin0(xp, degp, lin0_W, lin0_b, gcn_W):
    return pl.pallas_call(
        _tc_lin0_body,
        grid=(NB,),
        in_specs=[
            pl.BlockSpec((R, D), lambda i: (i, 0)),
            pl.BlockSpec((NC, R), lambda i: (0, i)),
            pl.BlockSpec((D, D), lambda i: (0, 0)),
            pl.BlockSpec((D,), lambda i: (0,)),
            pl.BlockSpec((D, D), lambda i: (0, 0)),
        ],
        out_specs=[
            pl.BlockSpec((R, D), lambda i: (i, 0)),
            pl.BlockSpec((R, D), lambda i: (i, 0)),
        ],
        out_shape=[
            jax.ShapeDtypeStruct((NP, D), _f32),
            jax.ShapeDtypeStruct((NP, D), _f32),
        ],
    )(xp, degp, lin0_W, lin0_b, gcn_W)


def _tc_round_body(aggp_ref, y_ref, h_ref, degp_ref, gb_ref,
                   wih_ref, whh_ref, bih_ref, bhh_ref, gw_ref,
                   hn_ref, yn_ref):
    agg = aggp_ref[0, :, :] + aggp_ref[1, :, :]
    deg = degp_ref[0, :] + degp_ref[1, :] + 1.0
    dinv = lax.rsqrt(deg)
    m = jnp.maximum(dinv[:, None] * (agg + y_ref[...]) + gb_ref[...], 0.0)
    h = h_ref[...]
    gi = _mm_t(m, wih_ref[...]) + bih_ref[...]
    gh = _mm_t(h, whh_ref[...]) + bhh_ref[...]
    r = jax.nn.sigmoid(gi[:, :D] + gh[:, :D])
    z = jax.nn.sigmoid(gi[:, D:2 * D] + gh[:, D:2 * D])
    n = jnp.tanh(gi[:, 2 * D:] + r * gh[:, 2 * D:])
    hn = (1.0 - z) * n + z * h
    hn_ref[...] = hn
    yn_ref[...] = dinv[:, None] * _mm_t(hn, gw_ref[...])


def _tc_round(aggp, y, h, degp, gcn_b, gru_Wih, gru_Whh, gru_bih, gru_bhh,
              gcn_W):
    return pl.pallas_call(
        _tc_round_body,
        grid=(NB,),
        in_specs=[
            pl.BlockSpec((NC, R, D), lambda i: (0, i, 0)),
            pl.BlockSpec((R, D), lambda i: (i, 0)),
            pl.BlockSpec((R, D), lambda i: (i, 0)),
            pl.BlockSpec((NC, R), lambda i: (0, i)),
            pl.BlockSpec((D,), lambda i: (0,)),
            pl.BlockSpec((3 * D, D), lambda i: (0, 0)),
            pl.BlockSpec((3 * D, D), lambda i: (0, 0)),
            pl.BlockSpec((3 * D,), lambda i: (0,)),
            pl.BlockSpec((3 * D,), lambda i: (0,)),
            pl.BlockSpec((D, D), lambda i: (0, 0)),
        ],
        out_specs=[
            pl.BlockSpec((R, D), lambda i: (i, 0)),
            pl.BlockSpec((R, D), lambda i: (i, 0)),
        ],
        out_shape=[
            jax.ShapeDtypeStruct((NP, D), _f32),
            jax.ShapeDtypeStruct((NP, D), _f32),
        ],
    )(aggp, y, h, degp, gcn_b, gru_Wih, gru_Whh, gru_bih, gru_bhh, gcn_W)


def _tc_set2set_body(x_ref, batch_ref, wih_ref, whh_ref, bih_ref, bhh_ref,
                     q_ref):
    x = x_ref[...]
    bvec = batch_ref[...]
    seg = lax.broadcasted_iota(jnp.int32, (N, B), 1)
    S = (bvec == seg).astype(_f32)

    h_l = jnp.zeros((B, D), _f32)
    c_l = jnp.zeros((B, D), _f32)
    q_star = jnp.zeros((B, 2 * D), _f32)
    for _ in range(3):
        gates = (_mm_t(q_star, wih_ref[...]) + bih_ref[...]
                 + _mm_t(h_l, whh_ref[...]) + bhh_ref[...])
        gi = gates[:, :D]
        gf = gates[:, D:2 * D]
        gg = gates[:, 2 * D:3 * D]
        go = gates[:, 3 * D:]
        c_l = jax.nn.sigmoid(gf) * c_l + jax.nn.sigmoid(gi) * jnp.tanh(gg)
        h_l = jax.nn.sigmoid(go) * jnp.tanh(c_l)
        qg = lax.dot_general(S, h_l, (((1,), (0,)), ((), ())),
                             preferred_element_type=_f32)
        e = jnp.sum(x * qg, axis=1, keepdims=True)
        em = jnp.where(S > 0.5, e, -1e30)
        mseg = jnp.max(em, axis=0, keepdims=True)
        mg = lax.dot_general(S, mseg, (((1,), (1,)), ((), ())),
                             preferred_element_type=_f32)
        e2 = jnp.exp(e - mg)
        sseg = lax.dot_general(S, e2, (((0,), (0,)), ((), ())),
                               preferred_element_type=_f32)
        sg = lax.dot_general(S, sseg, (((1,), (0,)), ((), ())),
                             preferred_element_type=_f32)
        a = e2 / (sg + 1e-16)
        r = lax.dot_general(S, a * x, (((0,), (0,)), ((), ())),
                            preferred_element_type=_f32)
        q_star = jnp.concatenate([h_l, r], axis=1)
    q_ref[...] = q_star


def _tc_set2set(x, batch2d, lstm_Wih, lstm_Whh, lstm_bih, lstm_bhh):
    return pl.pallas_call(
        _tc_set2set_body,
        out_shape=jax.ShapeDtypeStruct((B, 2 * D), _f32),
    )(x, batch2d, lstm_Wih, lstm_Whh, lstm_bih, lstm_bhh)


def kernel(x2, edge_index2, batch, lin0_W, lin0_b, gcn_W, gcn_b,
           gru_Wih, gru_Whh, gru_bih, gru_bhh,
           lstm_Wih, lstm_Whh, lstm_bih, lstm_bhh):
    src = edge_index2[0]
    dst = edge_index2[1]

    degp = _sc_degrees(dst).reshape(NC, NP)

    xp = jnp.pad(x2.astype(_f32), ((0, NP - N), (0, 0)))
    h, y = _tc_lin0(xp, degp, lin0_W, lin0_b, gcn_W)

    for _ in range(3):
        aggp = _sc_aggregate(y, src, dst).reshape(NC, NP, D)
        h, y = _tc_round(aggp, y, h, degp, gcn_b,
                         gru_Wih, gru_Whh, gru_bih, gru_bhh, gcn_W)

    feat_last = h[:N]
    batch2d = batch.reshape(N, 1)
    q_star = _tc_set2set(feat_last, batch2d,
                         lstm_Wih, lstm_Whh, lstm_bih, lstm_bhh)
    return (q_star, feat_last)


# speedup vs baseline: 24.1609x; 1.1395x over previous
import jax
import jax.numpy as jnp
from jax import lax
from jax.experimental import pallas as pl
from jax.experimental.pallas import tpu as pltpu
from jax.experimental.pallas import tpu_sc as plsc

N = 10000
E = 320000
D = 128
B = 64

NC = 2
NS = 16
NW = NC * NS
NP = 10240
RPW = NP // NS
EW = E // NW
K = 80
NCHUNK = EW // K

R = 1024
NB = NP // R

_f32 = jnp.float32


def _sc_deg_body(dst_hbm, degp_hbm, idx_v, ones_v, zero_v, acc_s, sem):
    c = lax.axis_index("c")
    s = lax.axis_index("s")
    w = c * NS + s

    one16 = jnp.ones((16,), _f32)
    zero16 = jnp.zeros((16,), _f32)
    for i in range(K // 16):
        ones_v[pl.ds(i * 16, 16)] = one16

    def zbody(i, carry):
        zero_v[pl.ds(i * 16, 16)] = zero16
        return carry

    lax.fori_loop(0, RPW // 16, zbody, 0)

    pltpu.sync_copy(zero_v, acc_s.at[pl.ds(s * RPW, RPW)])
    plsc.subcore_barrier()

    def body(j, carry):
        base = pl.multiple_of(w * EW + j * K, 8)
        pltpu.sync_copy(dst_hbm.at[pl.ds(base, K)], idx_v)
        pltpu.sync_copy(ones_v, acc_s.at[idx_v], add=True)
        return carry

    lax.fori_loop(0, NCHUNK, body, 0)
    plsc.subcore_barrier()

    pltpu.sync_copy(acc_s.at[pl.ds(s * RPW, RPW)],
                    degp_hbm.at[pl.ds(c * NP + s * RPW, RPW)])


def _sc_degrees(dst):
    mesh = plsc.VectorSubcoreMesh(core_axis_name="c", subcore_axis_name="s")
    fn = pl.kernel(
        _sc_deg_body,
        out_type=jax.ShapeDtypeStruct((NC * NP,), _f32),
        mesh=mesh,
        scratch_types=[
            pltpu.VMEM((K,), jnp.int32),
            pltpu.VMEM((K,), _f32),
            pltpu.VMEM((RPW,), _f32),
            pltpu.VMEM_SHARED((NP,), _f32),
            pltpu.SemaphoreType.DMA,
        ],
    )
    return fn(dst)


def _sc_agg_body(y_hbm, src_hbm, dst_hbm, aggp_hbm,
                 s0, s1, s2, s3, da, db, rows_a, rows_b, acc_s,
                 i0, i1, i2, i3, gsa, gsb, ea, eb):
    srcb = [s0, s1, s2, s3]
    isem = [i0, i1, i2, i3]
    dstb = [da, db]
    dsem = [ea, eb]
    rows = [rows_a, rows_b]
    gsem = [gsa, gsb]
    c = lax.axis_index("c")
    s = lax.axis_index("s")
    w = c * NS + s

    zero16 = jnp.zeros((16,), _f32)

    def zbody(i, carry):
        for cc in range(D // 16):
            rows_a[i, pl.ds(cc * 16, 16)] = zero16
        return carry

    lax.fori_loop(0, K, zbody, 0)
    for jj in range(RPW // K):
        pltpu.sync_copy(rows_a, acc_s.at[pl.ds(s * RPW + jj * K, K)])
    plsc.subcore_barrier()

    def ebase(j):
        return pl.multiple_of(w * EW + j * K, 8)

    def srcload(j, b):
        pltpu.async_copy(src_hbm.at[pl.ds(ebase(j), K)], srcb[b], isem[b])

    def issue(j, b, p):
        pltpu.make_async_copy(src_hbm.at[pl.ds(ebase(j), K)], srcb[b],
                              isem[b]).wait()
        pltpu.async_copy(y_hbm.at[srcb[b]], rows[p], gsem[p])
        pltpu.async_copy(dst_hbm.at[pl.ds(ebase(j), K)], dstb[p], dsem[p])

    def finish(j, b, p):
        pltpu.make_async_copy(y_hbm.at[srcb[b]], rows[p], gsem[p]).wait()
        pltpu.make_async_copy(dst_hbm.at[pl.ds(ebase(j), K)], dstb[p],
                              dsem[p]).wait()
        pltpu.sync_copy(rows[p], acc_s.at[dstb[p]], add=True)

    for t in range(3):
        srcload(t, t)
    issue(0, 0, 0)

    def body(g, carry):
        for bb in range(4):
            m = g * 4 + bb

            @pl.when(m + 3 < NCHUNK)
            def _():
                srcload(m + 3, (bb + 3) % 4)

            issue(m + 1, (bb + 1) % 4, (bb + 1) % 2)
            finish(m, bb, bb % 2)
        return carry

    lax.fori_loop(0, (NCHUNK - 1) // 4, body, 0)
    finish(NCHUNK - 1, (NCHUNK - 1) % 4, (NCHUNK - 1) % 2)
    plsc.subcore_barrier()

    pltpu.sync_copy(acc_s.at[pl.ds(s * RPW, RPW)],
                    aggp_hbm.at[pl.ds(c * NP + s * RPW, RPW)])


def _sc_aggregate(y, src, dst):
    mesh = plsc.VectorSubcoreMesh(core_axis_name="c", subcore_axis_name="s")
    fn = pl.kernel(
        _sc_agg_body,
        out_type=jax.ShapeDtypeStruct((NC * NP, D), _f32),
        mesh=mesh,
        scratch_types=(
            [pltpu.VMEM((K,), jnp.int32) for _ in range(6)]
            + [
                pltpu.VMEM((K, D), _f32),
                pltpu.VMEM((K, D), _f32),
                pltpu.VMEM_SHARED((NP, D), _f32),
            ]
            + [pltpu.SemaphoreType.DMA for _ in range(8)]
        ),
    )
    return fn(y, src, dst)


def _mm_t(a, w):
    return lax.dot_general(a, w, (((1,), (1,)), ((), ())),
                           preferred_element_type=_f32)


def _tc_lin0_body(x_ref, degp_ref, lw_ref, lb_ref, gw_ref, h_ref, y_ref):
    h = jnp.maximum(_mm_t(x_ref[...], lw_ref[...]) + lb_ref[...], 0.0)
    deg = degp_ref[0, :] + degp_ref[1, :] + 1.0
    dinv = lax.rsqrt(deg)
    h_ref[...] = h
    y_ref[...] = dinv[:, None] * _mm_t(h, gw_ref[...])


def _tc_lin0(xp, degp, lin0_W, lin0_b, gcn_W):
    return pl.pallas_call(
        _tc_lin0_body,
        grid=(NB,),
        in_specs=[
            pl.BlockSpec((R, D), lambda i: (i, 0)),
            pl.BlockSpec((NC, R), lambda i: (0, i)),
            pl.BlockSpec((D, D), lambda i: (0, 0)),
            pl.BlockSpec((D,), lambda i: (0,)),
            pl.BlockSpec((D, D), lambda i: (0, 0)),
        ],
        out_specs=[
            pl.BlockSpec((R, D), lambda i: (i, 0)),
            pl.BlockSpec((R, D), lambda i: (i, 0)),
        ],
        out_shape=[
            jax.ShapeDtypeStruct((NP, D), _f32),
            jax.ShapeDtypeStruct((NP, D), _f32),
        ],
    )(xp, degp, lin0_W, lin0_b, gcn_W)


def _tc_round_body(aggp_ref, y_ref, h_ref, degp_ref, gb_ref,
                   wih_ref, whh_ref, bih_ref, bhh_ref, gw_ref,
                   hn_ref, yn_ref):
    agg = aggp_ref[0, :, :] + aggp_ref[1, :, :]
    deg = degp_ref[0, :] + degp_ref[1, :] + 1.0
    dinv = lax.rsqrt(deg)
    m = jnp.maximum(dinv[:, None] * (agg + y_ref[...]) + gb_ref[...], 0.0)
    h = h_ref[...]
    gi = _mm_t(m, wih_ref[...]) + bih_ref[...]
    gh = _mm_t(h, whh_ref[...]) + bhh_ref[...]
    r = jax.nn.sigmoid(gi[:, :D] + gh[:, :D])
    z = jax.nn.sigmoid(gi[:, D:2 * D] + gh[:, D:2 * D])
    n = jnp.tanh(gi[:, 2 * D:] + r * gh[:, 2 * D:])
    hn = (1.0 - z) * n + z * h
    hn_ref[...] = hn
    yn_ref[...] = dinv[:, None] * _mm_t(hn, gw_ref[...])


def _tc_round(aggp, y, h, degp, gcn_b, gru_Wih, gru_Whh, gru_bih, gru_bhh,
              gcn_W):
    return pl.pallas_call(
        _tc_round_body,
        grid=(NB,),
        in_specs=[
            pl.BlockSpec((NC, R, D), lambda i: (0, i, 0)),
            pl.BlockSpec((R, D), lambda i: (i, 0)),
            pl.BlockSpec((R, D), lambda i: (i, 0)),
            pl.BlockSpec((NC, R), lambda i: (0, i)),
            pl.BlockSpec((D,), lambda i: (0,)),
            pl.BlockSpec((3 * D, D), lambda i: (0, 0)),
            pl.BlockSpec((3 * D, D), lambda i: (0, 0)),
            pl.BlockSpec((3 * D,), lambda i: (0,)),
            pl.BlockSpec((3 * D,), lambda i: (0,)),
            pl.BlockSpec((D, D), lambda i: (0, 0)),
        ],
        out_specs=[
            pl.BlockSpec((R, D), lambda i: (i, 0)),
            pl.BlockSpec((R, D), lambda i: (i, 0)),
        ],
        out_shape=[
            jax.ShapeDtypeStruct((NP, D), _f32),
            jax.ShapeDtypeStruct((NP, D), _f32),
        ],
    )(aggp, y, h, degp, gcn_b, gru_Wih, gru_Whh, gru_bih, gru_bhh, gcn_W)


def _tc_set2set_body(x_ref, batch_ref, wih_ref, whh_ref, bih_ref, bhh_ref,
                     q_ref):
    x = x_ref[...]
    bvec = batch_ref[...]
    seg = lax.broadcasted_iota(jnp.int32, (N, B), 1)
    S = (bvec == seg).astype(_f32)

    h_l = jnp.zeros((B, D), _f32)
    c_l = jnp.zeros((B, D), _f32)
    q_star = jnp.zeros((B, 2 * D), _f32)
    for _ in range(3):
        gates = (_mm_t(q_star, wih_ref[...]) + bih_ref[...]
                 + _mm_t(h_l, whh_ref[...]) + bhh_ref[...])
        gi = gates[:, :D]
        gf = gates[:, D:2 * D]
        gg = gates[:, 2 * D:3 * D]
        go = gates[:, 3 * D:]
        c_l = jax.nn.sigmoid(gf) * c_l + jax.nn.sigmoid(gi) * jnp.tanh(gg)
        h_l = jax.nn.sigmoid(go) * jnp.tanh(c_l)
        qg = lax.dot_general(S, h_l, (((1,), (0,)), ((), ())),
                             preferred_element_type=_f32)
        e = jnp.sum(x * qg, axis=1, keepdims=True)
        em = jnp.where(S > 0.5, e, -1e30)
        mseg = jnp.max(em, axis=0, keepdims=True)
        mg = lax.dot_general(S, mseg, (((1,), (1,)), ((), ())),
                             preferred_element_type=_f32)
        e2 = jnp.exp(e - mg)
        sseg = lax.dot_general(S, e2, (((0,), (0,)), ((), ())),
                               preferred_element_type=_f32)
        sg = lax.dot_general(S, sseg, (((1,), (0,)), ((), ())),
                             preferred_element_type=_f32)
        a = e2 / (sg + 1e-16)
        r = lax.dot_general(S, a * x, (((0,), (0,)), ((), ())),
                            preferred_element_type=_f32)
        q_star = jnp.concatenate([h_l, r], axis=1)
    q_ref[...] = q_star


def _tc_set2set(x, batch2d, lstm_Wih, lstm_Whh, lstm_bih, lstm_bhh):
    return pl.pallas_call(
        _tc_set2set_body,
        out_shape=jax.ShapeDtypeStruct((B, 2 * D), _f32),
    )(x, batch2d, lstm_Wih, lstm_Whh, lstm_bih, lstm_bhh)


def kernel(x2, edge_index2, batch, lin0_W, lin0_b, gcn_W, gcn_b,
           gru_Wih, gru_Whh, gru_bih, gru_bhh,
           lstm_Wih, lstm_Whh, lstm_bih, lstm_bhh):
    src = edge_index2[0]
    dst = edge_index2[1]

    degp = _sc_degrees(dst).reshape(NC, NP)

    xp = jnp.pad(x2.astype(_f32), ((0, NP - N), (0, 0)))
    h, y = _tc_lin0(xp, degp, lin0_W, lin0_b, gcn_W)

    for _ in range(3):
        aggp = _sc_aggregate(y, src, dst).reshape(NC, NP, D)
        h, y = _tc_round(aggp, y, h, degp, gcn_b,
                         gru_Wih, gru_Whh, gru_bih, gru_bhh, gcn_W)

    feat_last = h[:N]
    batch2d = batch.reshape(N, 1)
    q_star = _tc_set2set(feat_last, batch2d,
                         lstm_Wih, lstm_Whh, lstm_bih, lstm_bhh)
    return (q_star, feat_last)


# speedup vs baseline: 25.7015x; 1.0638x over previous
import jax
import jax.numpy as jnp
from jax import lax
from jax.experimental import pallas as pl
from jax.experimental.pallas import tpu as pltpu
from jax.experimental.pallas import tpu_sc as plsc

N = 10000
E = 320000
D = 128
B = 64

NC = 2
NS = 16
NW = NC * NS
NP = 10240
RPW = NP // NS
EW = E // NW
K = 80
NCHUNK = EW // K

R = 1024
NB = NP // R

_f32 = jnp.float32


def _sc_deg_body(dst_hbm, degp_hbm, ia, ib, ones_v, zero_v, acc_s, ma, mb):
    idxb = [ia, ib]
    dsem = [ma, mb]
    c = lax.axis_index("c")
    s = lax.axis_index("s")
    w = c * NS + s

    one16 = jnp.ones((16,), _f32)
    zero16 = jnp.zeros((16,), _f32)
    for i in range(K // 16):
        ones_v[pl.ds(i * 16, 16)] = one16

    def zbody(i, carry):
        zero_v[pl.ds(i * 16, 16)] = zero16
        return carry

    lax.fori_loop(0, RPW // 16, zbody, 0)

    pltpu.sync_copy(zero_v, acc_s.at[pl.ds(s * RPW, RPW)])
    plsc.subcore_barrier()

    def ebase(j):
        return pl.multiple_of(w * EW + j * K, 8)

    def idxload(j, p):
        pltpu.async_copy(dst_hbm.at[pl.ds(ebase(j), K)], idxb[p], dsem[p])

    def scat(j, p):
        pltpu.make_async_copy(dst_hbm.at[pl.ds(ebase(j), K)], idxb[p],
                              dsem[p]).wait()
        pltpu.sync_copy(ones_v, acc_s.at[idxb[p]], add=True)

    idxload(0, 0)

    def body(g, carry):
        j = 2 * g
        idxload(j + 1, 1)
        scat(j, 0)

        @pl.when(j + 2 < NCHUNK)
        def _():
            idxload(j + 2, 0)

        scat(j + 1, 1)
        return carry

    lax.fori_loop(0, (NCHUNK - 1) // 2, body, 0)
    scat(NCHUNK - 1, 0)
    plsc.subcore_barrier()

    pltpu.sync_copy(acc_s.at[pl.ds(s * RPW, RPW)],
                    degp_hbm.at[pl.ds(c * NP + s * RPW, RPW)])


def _sc_degrees(dst):
    mesh = plsc.VectorSubcoreMesh(core_axis_name="c", subcore_axis_name="s")
    fn = pl.kernel(
        _sc_deg_body,
        out_type=jax.ShapeDtypeStruct((NC * NP,), _f32),
        mesh=mesh,
        scratch_types=[
            pltpu.VMEM((K,), jnp.int32),
            pltpu.VMEM((K,), jnp.int32),
            pltpu.VMEM((K,), _f32),
            pltpu.VMEM((RPW,), _f32),
            pltpu.VMEM_SHARED((NP,), _f32),
            pltpu.SemaphoreType.DMA,
            pltpu.SemaphoreType.DMA,
        ],
    )
    return fn(dst)


def _sc_agg_body(y_hbm, src_hbm, dst_hbm, aggp_hbm,
                 s0, s1, s2, s3, da, db, rows_a, rows_b, acc_s,
                 i0, i1, i2, i3, gsa, gsb, ea, eb, ta, tb):
    srcb = [s0, s1, s2, s3]
    isem = [i0, i1, i2, i3]
    dstb = [da, db]
    dsem = [ea, eb]
    rows = [rows_a, rows_b]
    gsem = [gsa, gsb]
    ssem = [ta, tb]
    c = lax.axis_index("c")
    s = lax.axis_index("s")
    w = c * NS + s

    zero16 = jnp.zeros((16,), _f32)

    def zbody(i, carry):
        for cc in range(D // 16):
            rows_a[i, pl.ds(cc * 16, 16)] = zero16
        return carry

    lax.fori_loop(0, K, zbody, 0)
    for jj in range(RPW // K):
        pltpu.sync_copy(rows_a, acc_s.at[pl.ds(s * RPW + jj * K, K)])
    plsc.subcore_barrier()

    def ebase(j):
        return pl.multiple_of(w * EW + j * K, 8)

    def srcload(j, b):
        pltpu.async_copy(src_hbm.at[pl.ds(ebase(j), K)], srcb[b], isem[b])

    def issue(j, b, p):
        pltpu.make_async_copy(src_hbm.at[pl.ds(ebase(j), K)], srcb[b],
                              isem[b]).wait()
        pltpu.async_copy(y_hbm.at[srcb[b]], rows[p], gsem[p])
        pltpu.async_copy(dst_hbm.at[pl.ds(ebase(j), K)], dstb[p], dsem[p])

    def finish(j, b, p):
        pltpu.make_async_copy(y_hbm.at[srcb[b]], rows[p], gsem[p]).wait()
        pltpu.make_async_copy(dst_hbm.at[pl.ds(ebase(j), K)], dstb[p],
                              dsem[p]).wait()
        pltpu.async_copy(rows[p], acc_s.at[dstb[p]], ssem[p], add=True)

    def scatwait(p):
        pltpu.make_async_copy(rows[p], acc_s.at[dstb[p]], ssem[p]).wait()

    for t in range(3):
        srcload(t, t)
    issue(0, 0, 0)

    def body(g, carry):
        for bb in range(4):
            m = g * 4 + bb

            @pl.when(m + 3 < NCHUNK)
            def _():
                srcload(m + 3, (bb + 3) % 4)

            @pl.when(m >= 1)
            def _():
                scatwait((bb + 1) % 2)

            issue(m + 1, (bb + 1) % 4, (bb + 1) % 2)
            finish(m, bb, bb % 2)
        return carry

    lax.fori_loop(0, (NCHUNK - 1) // 4, body, 0)
    scatwait((NCHUNK - 2) % 2)
    finish(NCHUNK - 1, (NCHUNK - 1) % 4, (NCHUNK - 1) % 2)
    scatwait((NCHUNK - 1) % 2)
    plsc.subcore_barrier()

    pltpu.sync_copy(acc_s.at[pl.ds(s * RPW, RPW)],
                    aggp_hbm.at[pl.ds(c * NP + s * RPW, RPW)])


def _sc_aggregate(y, src, dst):
    mesh = plsc.VectorSubcoreMesh(core_axis_name="c", subcore_axis_name="s")
    fn = pl.kernel(
        _sc_agg_body,
        out_type=jax.ShapeDtypeStruct((NC * NP, D), _f32),
        mesh=mesh,
        scratch_types=(
            [pltpu.VMEM((K,), jnp.int32) for _ in range(6)]
            + [
                pltpu.VMEM((K, D), _f32),
                pltpu.VMEM((K, D), _f32),
                pltpu.VMEM_SHARED((NP, D), _f32),
            ]
            + [pltpu.SemaphoreType.DMA for _ in range(10)]
        ),
    )
    return fn(y, src, dst)


def _mm_t(a, w):
    return lax.dot_general(a, w, (((1,), (1,)), ((), ())),
                           preferred_element_type=_f32)


def _tc_lin0_body(x_ref, degp_ref, lw_ref, lb_ref, gw_ref, h_ref, y_ref):
    h = jnp.maximum(_mm_t(x_ref[...], lw_ref[...]) + lb_ref[...], 0.0)
    deg = degp_ref[0, :] + degp_ref[1, :] + 1.0
    dinv = lax.rsqrt(deg)
    h_ref[...] = h
    y_ref[...] = dinv[:, None] * _mm_t(h, gw_ref[...])


def _tc_lin0(xp, degp, lin0_W, lin0_b, gcn_W):
    return pl.pallas_call(
        _tc_lin0_body,
        grid=(NB,),
        in_specs=[
            pl.BlockSpec((R, D), lambda i: (i, 0)),
            pl.BlockSpec((NC, R), lambda i: (0, i)),
            pl.BlockSpec((D, D), lambda i: (0, 0)),
            pl.BlockSpec((D,), lambda i: (0,)),
            pl.BlockSpec((D, D), lambda i: (0, 0)),
        ],
        out_specs=[
            pl.BlockSpec((R, D), lambda i: (i, 0)),
            pl.BlockSpec((R, D), lambda i: (i, 0)),
        ],
        out_shape=[
            jax.ShapeDtypeStruct((NP, D), _f32),
            jax.ShapeDtypeStruct((NP, D), _f32),
        ],
    )(xp, degp, lin0_W, lin0_b, gcn_W)


def _tc_round_body(aggp_ref, y_ref, h_ref, degp_ref, gb_ref,
                   wih_ref, whh_ref, bih_ref, bhh_ref, gw_ref,
                   hn_ref, yn_ref):
    agg = aggp_ref[0, :, :] + aggp_ref[1, :, :]
    deg = degp_ref[0, :] + degp_ref[1, :] + 1.0
    dinv = lax.rsqrt(deg)
    m = jnp.maximum(dinv[:, None] * (agg + y_ref[...]) + gb_ref[...], 0.0)
    h = h_ref[...]
    gi = _mm_t(m, wih_ref[...]) + bih_ref[...]
    gh = _mm_t(h, whh_ref[...]) + bhh_ref[...]
    r = jax.nn.sigmoid(gi[:, :D] + gh[:, :D])
    z = jax.nn.sigmoid(gi[:, D:2 * D] + gh[:, D:2 * D])
    n = jnp.tanh(gi[:, 2 * D:] + r * gh[:, 2 * D:])
    hn = (1.0 - z) * n + z * h
    hn_ref[...] = hn
    yn_ref[...] = dinv[:, None] * _mm_t(hn, gw_ref[...])


def _tc_round(aggp, y, h, degp, gcn_b, gru_Wih, gru_Whh, gru_bih, gru_bhh,
              gcn_W):
    return pl.pallas_call(
        _tc_round_body,
        grid=(NB,),
        in_specs=[
            pl.BlockSpec((NC, R, D), lambda i: (0, i, 0)),
            pl.BlockSpec((R, D), lambda i: (i, 0)),
            pl.BlockSpec((R, D), lambda i: (i, 0)),
            pl.BlockSpec((NC, R), lambda i: (0, i)),
            pl.BlockSpec((D,), lambda i: (0,)),
            pl.BlockSpec((3 * D, D), lambda i: (0, 0)),
            pl.BlockSpec((3 * D, D), lambda i: (0, 0)),
            pl.BlockSpec((3 * D,), lambda i: (0,)),
            pl.BlockSpec((3 * D,), lambda i: (0,)),
            pl.BlockSpec((D, D), lambda i: (0, 0)),
        ],
        out_specs=[
            pl.BlockSpec((R, D), lambda i: (i, 0)),
            pl.BlockSpec((R, D), lambda i: (i, 0)),
        ],
        out_shape=[
            jax.ShapeDtypeStruct((NP, D), _f32),
            jax.ShapeDtypeStruct((NP, D), _f32),
        ],
    )(aggp, y, h, degp, gcn_b, gru_Wih, gru_Whh, gru_bih, gru_bhh, gcn_W)


def _tc_set2set_body(x_ref, batch_ref, wih_ref, whh_ref, bih_ref, bhh_ref,
                     q_ref):
    x = x_ref[...]
    bvec = batch_ref[...]
    seg = lax.broadcasted_iota(jnp.int32, (N, B), 1)
    S = (bvec == seg).astype(_f32)

    h_l = jnp.zeros((B, D), _f32)
    c_l = jnp.zeros((B, D), _f32)
    q_star = jnp.zeros((B, 2 * D), _f32)
    for _ in range(3):
        gates = (_mm_t(q_star, wih_ref[...]) + bih_ref[...]
                 + _mm_t(h_l, whh_ref[...]) + bhh_ref[...])
        gi = gates[:, :D]
        gf = gates[:, D:2 * D]
        gg = gates[:, 2 * D:3 * D]
        go = gates[:, 3 * D:]
        c_l = jax.nn.sigmoid(gf) * c_l + jax.nn.sigmoid(gi) * jnp.tanh(gg)
        h_l = jax.nn.sigmoid(go) * jnp.tanh(c_l)
        qg = lax.dot_general(S, h_l, (((1,), (0,)), ((), ())),
                             preferred_element_type=_f32)
        e = jnp.sum(x * qg, axis=1, keepdims=True)
        em = jnp.where(S > 0.5, e, -1e30)
        mseg = jnp.max(em, axis=0, keepdims=True)
        mg = lax.dot_general(S, mseg, (((1,), (1,)), ((), ())),
                             preferred_element_type=_f32)
        e2 = jnp.exp(e - mg)
        sseg = lax.dot_general(S, e2, (((0,), (0,)), ((), ())),
                               preferred_element_type=_f32)
        sg = lax.dot_general(S, sseg, (((1,), (0,)), ((), ())),
                             preferred_element_type=_f32)
        a = e2 / (sg + 1e-16)
        r = lax.dot_general(S, a * x, (((0,), (0,)), ((), ())),
                            preferred_element_type=_f32)
        q_star = jnp.concatenate([h_l, r], axis=1)
    q_ref[...] = q_star


def _tc_set2set(x, batch2d, lstm_Wih, lstm_Whh, lstm_bih, lstm_bhh):
    return pl.pallas_call(
        _tc_set2set_body,
        out_shape=jax.ShapeDtypeStruct((B, 2 * D), _f32),
    )(x, batch2d, lstm_Wih, lstm_Whh, lstm_bih, lstm_bhh)


def kernel(x2, edge_index2, batch, lin0_W, lin0_b, gcn_W, gcn_b,
           gru_Wih, gru_Whh, gru_bih, gru_bhh,
           lstm_Wih, lstm_Whh, lstm_bih, lstm_bhh):
    src = edge_index2[0]
    dst = edge_index2[1]

    degp = _sc_degrees(dst).reshape(NC, NP)

    xp = jnp.pad(x2.astype(_f32), ((0, NP - N), (0, 0)))
    h, y = _tc_lin0(xp, degp, lin0_W, lin0_b, gcn_W)

    for _ in range(3):
        aggp = _sc_aggregate(y, src, dst).reshape(NC, NP, D)
        h, y = _tc_round(aggp, y, h, degp, gcn_b,
                         gru_Wih, gru_Whh, gru_bih, gru_bhh, gcn_W)

    feat_last = h[:N]
    batch2d = batch.reshape(N, 1)
    q_star = _tc_set2set(feat_last, batch2d,
                         lstm_Wih, lstm_Whh, lstm_bih, lstm_bhh)
    return (q_star, feat_last)


# speedup vs baseline: 27.5030x; 1.0701x over previous
import jax
import jax.numpy as jnp
from jax import lax
from jax.experimental import pallas as pl
from jax.experimental.pallas import tpu as pltpu
from jax.experimental.pallas import tpu_sc as plsc

N = 10000
E = 320000
D = 128
B = 64

NC = 2
NS = 16
NW = NC * NS
NP = 10240
RPW = NP // NS
EW = E // NW
K = 80
NCHUNK = EW // K

R = 1024
NB = NP // R

_f32 = jnp.float32


def _sc_deg_body(dst_hbm, degp_hbm, ia, ib, ones_v, zero_v, acc_s, ma, mb):
    idxb = [ia, ib]
    dsem = [ma, mb]
    c = lax.axis_index("c")
    s = lax.axis_index("s")
    w = c * NS + s

    one16 = jnp.ones((16,), _f32)
    zero16 = jnp.zeros((16,), _f32)
    for i in range(K // 16):
        ones_v[pl.ds(i * 16, 16)] = one16

    def zbody(i, carry):
        zero_v[pl.ds(i * 16, 16)] = zero16
        return carry

    lax.fori_loop(0, RPW // 16, zbody, 0)

    pltpu.sync_copy(zero_v, acc_s.at[pl.ds(s * RPW, RPW)])
    plsc.subcore_barrier()

    def ebase(j):
        return pl.multiple_of(w * EW + j * K, 8)

    def idxload(j, p):
        pltpu.async_copy(dst_hbm.at[pl.ds(ebase(j), K)], idxb[p], dsem[p])

    def scat(j, p):
        pltpu.make_async_copy(dst_hbm.at[pl.ds(ebase(j), K)], idxb[p],
                              dsem[p]).wait()
        pltpu.sync_copy(ones_v, acc_s.at[idxb[p]], add=True)

    idxload(0, 0)

    def body(g, carry):
        j = 2 * g
        idxload(j + 1, 1)
        scat(j, 0)

        @pl.when(j + 2 < NCHUNK)
        def _():
            idxload(j + 2, 0)

        scat(j + 1, 1)
        return carry

    lax.fori_loop(0, (NCHUNK - 1) // 2, body, 0)
    scat(NCHUNK - 1, 0)
    plsc.subcore_barrier()

    pltpu.sync_copy(acc_s.at[pl.ds(s * RPW, RPW)],
                    degp_hbm.at[pl.ds(c * NP + s * RPW, RPW)])


def _sc_degrees(dst):
    mesh = plsc.VectorSubcoreMesh(core_axis_name="c", subcore_axis_name="s")
    fn = pl.kernel(
        _sc_deg_body,
        out_type=jax.ShapeDtypeStruct((NC * NP,), _f32),
        mesh=mesh,
        scratch_types=[
            pltpu.VMEM((K,), jnp.int32),
            pltpu.VMEM((K,), jnp.int32),
            pltpu.VMEM((K,), _f32),
            pltpu.VMEM((RPW,), _f32),
            pltpu.VMEM_SHARED((NP,), _f32),
            pltpu.SemaphoreType.DMA,
            pltpu.SemaphoreType.DMA,
        ],
    )
    return fn(dst)


KA = 128
NFULL = EW // KA
KT = EW - NFULL * KA


def _sc_agg_body(y_hbm, src_hbm, dst_hbm, aggp_hbm,
                 s0, s1, da, db, st, dt, rows_a, rows_b, rows_t, acc_s,
                 i0, i1, ea, eb, gsa, gsb, ta, tb, tt):
    srcb = [s0, s1]
    dstb = [da, db]
    isem = [i0, i1]
    dsem = [ea, eb]
    rows = [rows_a, rows_b]
    gsem = [gsa, gsb]
    ssem = [ta, tb]
    c = lax.axis_index("c")
    s = lax.axis_index("s")
    w = c * NS + s

    zero16 = jnp.zeros((16,), _f32)

    def zbody(i, carry):
        for cc in range(D // 16):
            rows_a[i, pl.ds(cc * 16, 16)] = zero16
        return carry

    lax.fori_loop(0, KA, zbody, 0)
    for jj in range(RPW // KA):
        pltpu.sync_copy(rows_a, acc_s.at[pl.ds(s * RPW + jj * KA, KA)])
    plsc.subcore_barrier()

    def ebase(j):
        return pl.multiple_of(w * EW + j * KA, 8)

    def srcload(j, b):
        pltpu.async_copy(src_hbm.at[pl.ds(ebase(j), KA)], srcb[b], isem[b])

    def issue(j, p):
        pltpu.make_async_copy(src_hbm.at[pl.ds(ebase(j), KA)], srcb[p],
                              isem[p]).wait()
        pltpu.async_copy(y_hbm.at[srcb[p]], rows[p], gsem[p])
        pltpu.async_copy(dst_hbm.at[pl.ds(ebase(j), KA)], dstb[p], dsem[p])

    def finish(j, p):
        pltpu.make_async_copy(y_hbm.at[srcb[p]], rows[p], gsem[p]).wait()
        pltpu.make_async_copy(dst_hbm.at[pl.ds(ebase(j), KA)], dstb[p],
                              dsem[p]).wait()
        pltpu.async_copy(rows[p], acc_s.at[dstb[p]], ssem[p], add=True)

    def scatwait(p):
        pltpu.make_async_copy(rows[p], acc_s.at[dstb[p]], ssem[p]).wait()

    srcload(0, 0)
    srcload(1, 1)
    issue(0, 0)

    def body(g, carry):
        for bb in range(2):
            m = g * 2 + bb
            op = (bb + 1) % 2

            @pl.when(m >= 1)
            def _():
                scatwait(op)

            @pl.when(m + 1 < NFULL)
            def _():
                issue(m + 1, op)

            finish(m, bb)

            @pl.when(m + 2 < NFULL)
            def _():
                srcload(m + 2, bb)
        return carry

    lax.fori_loop(0, NFULL // 2, body, 0)
    tbase = pl.multiple_of(w * EW + NFULL * KA, 8)
    pltpu.sync_copy(src_hbm.at[pl.ds(tbase, KT)], st)
    pltpu.async_copy(y_hbm.at[st], rows_t, tt).wait()
    pltpu.sync_copy(dst_hbm.at[pl.ds(tbase, KT)], dt)
    pltpu.sync_copy(rows_t, acc_s.at[dt], add=True)
    scatwait((NFULL - 1) % 2)
    plsc.subcore_barrier()

    pltpu.sync_copy(acc_s.at[pl.ds(s * RPW, RPW)],
                    aggp_hbm.at[pl.ds(c * NP + s * RPW, RPW)])


def _sc_aggregate(y, src, dst):
    mesh = plsc.VectorSubcoreMesh(core_axis_name="c", subcore_axis_name="s")
    fn = pl.kernel(
        _sc_agg_body,
        out_type=jax.ShapeDtypeStruct((NC * NP, D), _f32),
        mesh=mesh,
        scratch_types=[
            pltpu.VMEM((KA,), jnp.int32),
            pltpu.VMEM((KA,), jnp.int32),
            pltpu.VMEM((KA,), jnp.int32),
            pltpu.VMEM((KA,), jnp.int32),
            pltpu.VMEM((KT,), jnp.int32),
            pltpu.VMEM((KT,), jnp.int32),
            pltpu.VMEM((KA, D), _f32),
            pltpu.VMEM((KA, D), _f32),
            pltpu.VMEM((KT, D), _f32),
            pltpu.VMEM_SHARED((NP, D), _f32),
        ] + [pltpu.SemaphoreType.DMA for _ in range(9)],
    )
    return fn(y, src, dst)


def _mm_t(a, w):
    return lax.dot_general(a, w, (((1,), (1,)), ((), ())),
                           preferred_element_type=_f32)


def _tc_lin0_body(x_ref, degp_ref, lw_ref, lb_ref, gw_ref, h_ref, y_ref):
    h = jnp.maximum(_mm_t(x_ref[...], lw_ref[...]) + lb_ref[...], 0.0)
    deg = degp_ref[0, :] + degp_ref[1, :] + 1.0
    dinv = lax.rsqrt(deg)
    h_ref[...] = h
    y_ref[...] = dinv[:, None] * _mm_t(h, gw_ref[...])


def _tc_lin0(xp, degp, lin0_W, lin0_b, gcn_W):
    return pl.pallas_call(
        _tc_lin0_body,
        grid=(NB,),
        in_specs=[
            pl.BlockSpec((R, D), lambda i: (i, 0)),
            pl.BlockSpec((NC, R), lambda i: (0, i)),
            pl.BlockSpec((D, D), lambda i: (0, 0)),
            pl.BlockSpec((D,), lambda i: (0,)),
            pl.BlockSpec((D, D), lambda i: (0, 0)),
        ],
        out_specs=[
            pl.BlockSpec((R, D), lambda i: (i, 0)),
            pl.BlockSpec((R, D), lambda i: (i, 0)),
        ],
        out_shape=[
            jax.ShapeDtypeStruct((NP, D), _f32),
            jax.ShapeDtypeStruct((NP, D), _f32),
        ],
    )(xp, degp, lin0_W, lin0_b, gcn_W)


def _tc_round_body(aggp_ref, y_ref, h_ref, degp_ref, gb_ref,
                   wih_ref, whh_ref, bih_ref, bhh_ref, gw_ref,
                   hn_ref, yn_ref):
    agg = aggp_ref[0, :, :] + aggp_ref[1, :, :]
    deg = degp_ref[0, :] + degp_ref[1, :] + 1.0
    dinv = lax.rsqrt(deg)
    m = jnp.maximum(dinv[:, None] * (agg + y_ref[...]) + gb_ref[...], 0.0)
    h = h_ref[...]
    gi = _mm_t(m, wih_ref[...]) + bih_ref[...]
    gh = _mm_t(h, whh_ref[...]) + bhh_ref[...]
    r = jax.nn.sigmoid(gi[:, :D] + gh[:, :D])
    z = jax.nn.sigmoid(gi[:, D:2 * D] + gh[:, D:2 * D])
    n = jnp.tanh(gi[:, 2 * D:] + r * gh[:, 2 * D:])
    hn = (1.0 - z) * n + z * h
    hn_ref[...] = hn
    yn_ref[...] = dinv[:, None] * _mm_t(hn, gw_ref[...])


def _tc_round(aggp, y, h, degp, gcn_b, gru_Wih, gru_Whh, gru_bih, gru_bhh,
              gcn_W):
    return pl.pallas_call(
        _tc_round_body,
        grid=(NB,),
        in_specs=[
            pl.BlockSpec((NC, R, D), lambda i: (0, i, 0)),
            pl.BlockSpec((R, D), lambda i: (i, 0)),
            pl.BlockSpec((R, D), lambda i: (i, 0)),
            pl.BlockSpec((NC, R), lambda i: (0, i)),
            pl.BlockSpec((D,), lambda i: (0,)),
            pl.BlockSpec((3 * D, D), lambda i: (0, 0)),
            pl.BlockSpec((3 * D, D), lambda i: (0, 0)),
            pl.BlockSpec((3 * D,), lambda i: (0,)),
            pl.BlockSpec((3 * D,), lambda i: (0,)),
            pl.BlockSpec((D, D), lambda i: (0, 0)),
        ],
        out_specs=[
            pl.BlockSpec((R, D), lambda i: (i, 0)),
            pl.BlockSpec((R, D), lambda i: (i, 0)),
        ],
        out_shape=[
            jax.ShapeDtypeStruct((NP, D), _f32),
            jax.ShapeDtypeStruct((NP, D), _f32),
        ],
    )(aggp, y, h, degp, gcn_b, gru_Wih, gru_Whh, gru_bih, gru_bhh, gcn_W)


def _tc_set2set_body(x_ref, batch_ref, wih_ref, whh_ref, bih_ref, bhh_ref,
                     q_ref):
    x = x_ref[...]
    bvec = batch_ref[...]
    seg = lax.broadcasted_iota(jnp.int32, (N, B), 1)
    S = (bvec == seg).astype(_f32)

    h_l = jnp.zeros((B, D), _f32)
    c_l = jnp.zeros((B, D), _f32)
    q_star = jnp.zeros((B, 2 * D), _f32)
    for _ in range(3):
        gates = (_mm_t(q_star, wih_ref[...]) + bih_ref[...]
                 + _mm_t(h_l, whh_ref[...]) + bhh_ref[...])
        gi = gates[:, :D]
        gf = gates[:, D:2 * D]
        gg = gates[:, 2 * D:3 * D]
        go = gates[:, 3 * D:]
        c_l = jax.nn.sigmoid(gf) * c_l + jax.nn.sigmoid(gi) * jnp.tanh(gg)
        h_l = jax.nn.sigmoid(go) * jnp.tanh(c_l)
        qg = lax.dot_general(S, h_l, (((1,), (0,)), ((), ())),
                             preferred_element_type=_f32)
        e = jnp.sum(x * qg, axis=1, keepdims=True)
        em = jnp.where(S > 0.5, e, -1e30)
        mseg = jnp.max(em, axis=0, keepdims=True)
        mg = lax.dot_general(S, mseg, (((1,), (1,)), ((), ())),
                             preferred_element_type=_f32)
        e2 = jnp.exp(e - mg)
        sseg = lax.dot_general(S, e2, (((0,), (0,)), ((), ())),
                               preferred_element_type=_f32)
        sg = lax.dot_general(S, sseg, (((1,), (0,)), ((), ())),
                             preferred_element_type=_f32)
        a = e2 / (sg + 1e-16)
        r = lax.dot_general(S, a * x, (((0,), (0,)), ((), ())),
                            preferred_element_type=_f32)
        q_star = jnp.concatenate([h_l, r], axis=1)
    q_ref[...] = q_star


def _tc_set2set(x, batch2d, lstm_Wih, lstm_Whh, lstm_bih, lstm_bhh):
    return pl.pallas_call(
        _tc_set2set_body,
        out_shape=jax.ShapeDtypeStruct((B, 2 * D), _f32),
    )(x, batch2d, lstm_Wih, lstm_Whh, lstm_bih, lstm_bhh)


def kernel(x2, edge_index2, batch, lin0_W, lin0_b, gcn_W, gcn_b,
           gru_Wih, gru_Whh, gru_bih, gru_bhh,
           lstm_Wih, lstm_Whh, lstm_bih, lstm_bhh):
    src = edge_index2[0]
    dst = edge_index2[1]

    degp = _sc_degrees(dst).reshape(NC, NP)

    xp = jnp.pad(x2.astype(_f32), ((0, NP - N), (0, 0)))
    h, y = _tc_lin0(xp, degp, lin0_W, lin0_b, gcn_W)

    for _ in range(3):
        aggp = _sc_aggregate(y, src, dst).reshape(NC, NP, D)
        h, y = _tc_round(aggp, y, h, degp, gcn_b,
                         gru_Wih, gru_Whh, gru_bih, gru_bhh, gcn_W)

    feat_last = h[:N]
    batch2d = batch.reshape(N, 1)
    q_star = _tc_set2set(feat_last, batch2d,
                         lstm_Wih, lstm_Whh, lstm_bih, lstm_bhh)
    return (q_star, feat_last)


# speedup vs baseline: 28.0030x; 1.0182x over previous
import jax
import jax.numpy as jnp
from jax import lax
from jax.experimental import pallas as pl
from jax.experimental.pallas import tpu as pltpu
from jax.experimental.pallas import tpu_sc as plsc

N = 10000
E = 320000
D = 128
B = 64

NC = 2
NS = 16
NW = NC * NS
NP = 10240
RPW = NP // NS
EW = E // NW
R = 1024
NB = NP // R

_f32 = jnp.float32


KA = 128
NFULL = EW // KA
KT = EW - NFULL * KA


def _sc_deg_body(dst_hbm, degp_hbm, ia, ib, it, ones_v, zero_v, acc_s,
                 ma, mb, mt):
    idxb = [ia, ib]
    dsem = [ma, mb]
    c = lax.axis_index("c")
    s = lax.axis_index("s")
    w = c * NS + s

    one16 = jnp.ones((16,), _f32)
    zero16 = jnp.zeros((16,), _f32)
    for i in range(KA // 16):
        ones_v[pl.ds(i * 16, 16)] = one16

    def zbody(i, carry):
        zero_v[pl.ds(i * 16, 16)] = zero16
        return carry

    lax.fori_loop(0, RPW // 16, zbody, 0)

    pltpu.sync_copy(zero_v, acc_s.at[pl.ds(s * RPW, RPW)])
    plsc.subcore_barrier()

    def ebase(j):
        return pl.multiple_of(w * EW + j * KA, 8)

    def idxload(j, p):
        pltpu.async_copy(dst_hbm.at[pl.ds(ebase(j), KA)], idxb[p], dsem[p])

    def scat(j, p):
        pltpu.make_async_copy(dst_hbm.at[pl.ds(ebase(j), KA)], idxb[p],
                              dsem[p]).wait()
        pltpu.sync_copy(ones_v.at[pl.ds(0, KA)], acc_s.at[idxb[p]],
                        add=True)

    idxload(0, 0)

    def body(g, carry):
        j = 2 * g
        idxload(j + 1, 1)
        scat(j, 0)

        @pl.when(j + 2 < NFULL)
        def _():
            idxload(j + 2, 0)

        scat(j + 1, 1)
        return carry

    lax.fori_loop(0, NFULL // 2, body, 0)
    tbase = pl.multiple_of(w * EW + NFULL * KA, 8)
    pltpu.sync_copy(dst_hbm.at[pl.ds(tbase, KT)], it)
    pltpu.sync_copy(ones_v.at[pl.ds(0, KT)], acc_s.at[it], add=True)
    plsc.subcore_barrier()

    pltpu.sync_copy(acc_s.at[pl.ds(s * RPW, RPW)],
                    degp_hbm.at[pl.ds(c * NP + s * RPW, RPW)])


def _sc_degrees(dst):
    mesh = plsc.VectorSubcoreMesh(core_axis_name="c", subcore_axis_name="s")
    fn = pl.kernel(
        _sc_deg_body,
        out_type=jax.ShapeDtypeStruct((NC * NP,), _f32),
        mesh=mesh,
        scratch_types=[
            pltpu.VMEM((KA,), jnp.int32),
            pltpu.VMEM((KA,), jnp.int32),
            pltpu.VMEM((KT,), jnp.int32),
            pltpu.VMEM((KA,), _f32),
            pltpu.VMEM((RPW,), _f32),
            pltpu.VMEM_SHARED((NP,), _f32),
            pltpu.SemaphoreType.DMA,
            pltpu.SemaphoreType.DMA,
            pltpu.SemaphoreType.DMA,
        ],
    )
    return fn(dst)


def _sc_agg_body(y_hbm, src_hbm, dst_hbm, aggp_hbm,
                 s0, s1, da, db, st, dt, rows_a, rows_b, rows_t, acc_s,
                 i0, i1, ea, eb, gsa, gsb, ta, tb, tt):
    srcb = [s0, s1]
    dstb = [da, db]
    isem = [i0, i1]
    dsem = [ea, eb]
    rows = [rows_a, rows_b]
    gsem = [gsa, gsb]
    ssem = [ta, tb]
    c = lax.axis_index("c")
    s = lax.axis_index("s")
    w = c * NS + s

    zero16 = jnp.zeros((16,), _f32)

    def zbody(i, carry):
        for cc in range(D // 16):
            rows_a[i, pl.ds(cc * 16, 16)] = zero16
        return carry

    lax.fori_loop(0, KA, zbody, 0)
    for jj in range(RPW // KA):
        pltpu.sync_copy(rows_a, acc_s.at[pl.ds(s * RPW + jj * KA, KA)])
    plsc.subcore_barrier()

    def ebase(j):
        return pl.multiple_of(w * EW + j * KA, 8)

    def srcload(j, b):
        pltpu.async_copy(src_hbm.at[pl.ds(ebase(j), KA)], srcb[b], isem[b])

    def issue(j, p):
        pltpu.make_async_copy(src_hbm.at[pl.ds(ebase(j), KA)], srcb[p],
                              isem[p]).wait()
        pltpu.async_copy(y_hbm.at[srcb[p]], rows[p], gsem[p])
        pltpu.async_copy(dst_hbm.at[pl.ds(ebase(j), KA)], dstb[p], dsem[p])

    def finish(j, p):
        pltpu.make_async_copy(y_hbm.at[srcb[p]], rows[p], gsem[p]).wait()
        pltpu.make_async_copy(dst_hbm.at[pl.ds(ebase(j), KA)], dstb[p],
                              dsem[p]).wait()
        pltpu.async_copy(rows[p], acc_s.at[dstb[p]], ssem[p], add=True)

    def scatwait(p):
        pltpu.make_async_copy(rows[p], acc_s.at[dstb[p]], ssem[p]).wait()

    srcload(0, 0)
    srcload(1, 1)
    issue(0, 0)

    def body(g, carry):
        for bb in range(2):
            m = g * 2 + bb
            op = (bb + 1) % 2

            @pl.when(m >= 1)
            def _():
                scatwait(op)

            @pl.when(m + 1 < NFULL)
            def _():
                issue(m + 1, op)

            finish(m, bb)

            @pl.when(m + 2 < NFULL)
            def _():
                srcload(m + 2, bb)
        return carry

    lax.fori_loop(0, NFULL // 2, body, 0)
    tbase = pl.multiple_of(w * EW + NFULL * KA, 8)
    pltpu.sync_copy(src_hbm.at[pl.ds(tbase, KT)], st)
    pltpu.async_copy(y_hbm.at[st], rows_t, tt).wait()
    pltpu.sync_copy(dst_hbm.at[pl.ds(tbase, KT)], dt)
    pltpu.sync_copy(rows_t, acc_s.at[dt], add=True)
    scatwait((NFULL - 1) % 2)
    plsc.subcore_barrier()

    pltpu.sync_copy(acc_s.at[pl.ds(s * RPW, RPW)],
                    aggp_hbm.at[pl.ds(c * NP + s * RPW, RPW)])


def _sc_aggregate(y, src, dst):
    mesh = plsc.VectorSubcoreMesh(core_axis_name="c", subcore_axis_name="s")
    fn = pl.kernel(
        _sc_agg_body,
        out_type=jax.ShapeDtypeStruct((NC * NP, D), _f32),
        mesh=mesh,
        scratch_types=[
            pltpu.VMEM((KA,), jnp.int32),
            pltpu.VMEM((KA,), jnp.int32),
            pltpu.VMEM((KA,), jnp.int32),
            pltpu.VMEM((KA,), jnp.int32),
            pltpu.VMEM((KT,), jnp.int32),
            pltpu.VMEM((KT,), jnp.int32),
            pltpu.VMEM((KA, D), _f32),
            pltpu.VMEM((KA, D), _f32),
            pltpu.VMEM((KT, D), _f32),
            pltpu.VMEM_SHARED((NP, D), _f32),
        ] + [pltpu.SemaphoreType.DMA for _ in range(9)],
    )
    return fn(y, src, dst)


def _mm_t(a, w):
    return lax.dot_general(a, w, (((1,), (1,)), ((), ())),
                           preferred_element_type=_f32)


def _tc_lin0_body(x_ref, degp_ref, lw_ref, lb_ref, gw_ref, h_ref, y_ref):
    h = jnp.maximum(_mm_t(x_ref[...], lw_ref[...]) + lb_ref[...], 0.0)
    deg = degp_ref[0, :] + degp_ref[1, :] + 1.0
    dinv = lax.rsqrt(deg)
    h_ref[...] = h
    y_ref[...] = dinv[:, None] * _mm_t(h, gw_ref[...])


def _tc_lin0(xp, degp, lin0_W, lin0_b, gcn_W):
    return pl.pallas_call(
        _tc_lin0_body,
        grid=(NB,),
        in_specs=[
            pl.BlockSpec((R, D), lambda i: (i, 0)),
            pl.BlockSpec((NC, R), lambda i: (0, i)),
            pl.BlockSpec((D, D), lambda i: (0, 0)),
            pl.BlockSpec((D,), lambda i: (0,)),
            pl.BlockSpec((D, D), lambda i: (0, 0)),
        ],
        out_specs=[
            pl.BlockSpec((R, D), lambda i: (i, 0)),
            pl.BlockSpec((R, D), lambda i: (i, 0)),
        ],
        out_shape=[
            jax.ShapeDtypeStruct((NP, D), _f32),
            jax.ShapeDtypeStruct((NP, D), _f32),
        ],
    )(xp, degp, lin0_W, lin0_b, gcn_W)


def _tc_round_body(aggp_ref, y_ref, h_ref, degp_ref, gb_ref,
                   wih_ref, whh_ref, bih_ref, bhh_ref, gw_ref,
                   hn_ref, yn_ref):
    agg = aggp_ref[0, :, :] + aggp_ref[1, :, :]
    deg = degp_ref[0, :] + degp_ref[1, :] + 1.0
    dinv = lax.rsqrt(deg)
    m = jnp.maximum(dinv[:, None] * (agg + y_ref[...]) + gb_ref[...], 0.0)
    h = h_ref[...]
    gi = _mm_t(m, wih_ref[...]) + bih_ref[...]
    gh = _mm_t(h, whh_ref[...]) + bhh_ref[...]
    r = jax.nn.sigmoid(gi[:, :D] + gh[:, :D])
    z = jax.nn.sigmoid(gi[:, D:2 * D] + gh[:, D:2 * D])
    n = jnp.tanh(gi[:, 2 * D:] + r * gh[:, 2 * D:])
    hn = (1.0 - z) * n + z * h
    hn_ref[...] = hn
    yn_ref[...] = dinv[:, None] * _mm_t(hn, gw_ref[...])


def _tc_round(aggp, y, h, degp, gcn_b, gru_Wih, gru_Whh, gru_bih, gru_bhh,
              gcn_W):
    return pl.pallas_call(
        _tc_round_body,
        grid=(NB,),
        in_specs=[
            pl.BlockSpec((NC, R, D), lambda i: (0, i, 0)),
            pl.BlockSpec((R, D), lambda i: (i, 0)),
            pl.BlockSpec((R, D), lambda i: (i, 0)),
            pl.BlockSpec((NC, R), lambda i: (0, i)),
            pl.BlockSpec((D,), lambda i: (0,)),
            pl.BlockSpec((3 * D, D), lambda i: (0, 0)),
            pl.BlockSpec((3 * D, D), lambda i: (0, 0)),
            pl.BlockSpec((3 * D,), lambda i: (0,)),
            pl.BlockSpec((3 * D,), lambda i: (0,)),
            pl.BlockSpec((D, D), lambda i: (0, 0)),
        ],
        out_specs=[
            pl.BlockSpec((R, D), lambda i: (i, 0)),
            pl.BlockSpec((R, D), lambda i: (i, 0)),
        ],
        out_shape=[
            jax.ShapeDtypeStruct((NP, D), _f32),
            jax.ShapeDtypeStruct((NP, D), _f32),
        ],
    )(aggp, y, h, degp, gcn_b, gru_Wih, gru_Whh, gru_bih, gru_bhh, gcn_W)


def _tc_set2set_body(x_ref, batch_ref, wih_ref, whh_ref, bih_ref, bhh_ref,
                     q_ref):
    x = x_ref[...]
    bvec = batch_ref[...]
    seg = lax.broadcasted_iota(jnp.int32, (N, B), 1)
    S = (bvec == seg).astype(_f32)

    h_l = jnp.zeros((B, D), _f32)
    c_l = jnp.zeros((B, D), _f32)
    q_star = jnp.zeros((B, 2 * D), _f32)
    for _ in range(3):
        gates = (_mm_t(q_star, wih_ref[...]) + bih_ref[...]
                 + _mm_t(h_l, whh_ref[...]) + bhh_ref[...])
        gi = gates[:, :D]
        gf = gates[:, D:2 * D]
        gg = gates[:, 2 * D:3 * D]
        go = gates[:, 3 * D:]
        c_l = jax.nn.sigmoid(gf) * c_l + jax.nn.sigmoid(gi) * jnp.tanh(gg)
        h_l = jax.nn.sigmoid(go) * jnp.tanh(c_l)
        qg = lax.dot_general(S, h_l, (((1,), (0,)), ((), ())),
                             preferred_element_type=_f32)
        e = jnp.sum(x * qg, axis=1, keepdims=True)
        em = jnp.where(S > 0.5, e, -1e30)
        mseg = jnp.max(em, axis=0, keepdims=True)
        mg = lax.dot_general(S, mseg, (((1,), (1,)), ((), ())),
                             preferred_element_type=_f32)
        e2 = jnp.exp(e - mg)
        sseg = lax.dot_general(S, e2, (((0,), (0,)), ((), ())),
                               preferred_element_type=_f32)
        sg = lax.dot_general(S, sseg, (((1,), (0,)), ((), ())),
                             preferred_element_type=_f32)
        a = e2 / (sg + 1e-16)
        r = lax.dot_general(S, a * x, (((0,), (0,)), ((), ())),
                            preferred_element_type=_f32)
        q_star = jnp.concatenate([h_l, r], axis=1)
    q_ref[...] = q_star


def _tc_set2set(x, batch2d, lstm_Wih, lstm_Whh, lstm_bih, lstm_bhh):
    return pl.pallas_call(
        _tc_set2set_body,
        out_shape=jax.ShapeDtypeStruct((B, 2 * D), _f32),
    )(x, batch2d, lstm_Wih, lstm_Whh, lstm_bih, lstm_bhh)


def kernel(x2, edge_index2, batch, lin0_W, lin0_b, gcn_W, gcn_b,
           gru_Wih, gru_Whh, gru_bih, gru_bhh,
           lstm_Wih, lstm_Whh, lstm_bih, lstm_bhh):
    src = edge_index2[0]
    dst = edge_index2[1]

    degp = _sc_degrees(dst).reshape(NC, NP)

    xp = jnp.pad(x2.astype(_f32), ((0, NP - N), (0, 0)))
    h, y = _tc_lin0(xp, degp, lin0_W, lin0_b, gcn_W)

    for _ in range(3):
        aggp = _sc_aggregate(y, src, dst).reshape(NC, NP, D)
        h, y = _tc_round(aggp, y, h, degp, gcn_b,
                         gru_Wih, gru_Whh, gru_bih, gru_bhh, gcn_W)

    feat_last = h[:N]
    batch2d = batch.reshape(N, 1)
    q_star = _tc_set2set(feat_last, batch2d,
                         lstm_Wih, lstm_Whh, lstm_bih, lstm_bhh)
    return (q_star, feat_last)
